# baseline (speedup 1.0000x reference)
import numpy as np
import jax
import jax.numpy as jnp
from jax.experimental import pallas as pl
from jax.experimental.pallas import tpu as pltpu

_NUM_T = 1000

_beta = np.linspace(0.0001, 0.02, _NUM_T).astype(np.float32)
_ac = np.cumprod((1.0 - _beta).astype(np.float32), dtype=np.float32)
_TABLE_A = np.sqrt(_ac).astype(np.float32)
_TABLE_B = np.sqrt(1.0 - _ac).astype(np.float32)

_B = 128
_ROWS = 192
_COLS = 1024
_SPLIT = 2
_CH = _ROWS // _SPLIT


def _body(t_ref, ta_ref, tb_ref, x_ref, n_ref, o_ref):
    b = pl.program_id(0)
    ti = t_ref[b]
    a = ta_ref[ti]
    c = tb_ref[ti]
    o_ref[...] = a * x_ref[...] + c * n_ref[...]


def kernel(x_start, t, noise):
    xf = x_start.reshape(_B, _ROWS, _COLS)
    nf = noise.reshape(_B, _ROWS, _COLS)
    ti = t.astype(jnp.int32)
    ta = jnp.asarray(_TABLE_A)
    tb = jnp.asarray(_TABLE_B)

    grid_spec = pltpu.PrefetchScalarGridSpec(
        num_scalar_prefetch=3,
        grid=(_B, _SPLIT),
        in_specs=[
            pl.BlockSpec((1, _CH, _COLS), lambda b, j, *_: (b, j, 0)),
            pl.BlockSpec((1, _CH, _COLS), lambda b, j, *_: (b, j, 0)),
        ],
        out_specs=pl.BlockSpec((1, _CH, _COLS), lambda b, j, *_: (b, j, 0)),
    )
    out = pl.pallas_call(
        _body,
        grid_spec=grid_spec,
        out_shape=jax.ShapeDtypeStruct((_B, _ROWS, _COLS), jnp.float32),
    )(ti, ta, tb, xf, nf)
    return out.reshape(x_start.shape)


# speedup vs baseline: 1.0041x; 1.0041x over previous
import numpy as np
import jax
import jax.numpy as jnp
from jax.experimental import pallas as pl
from jax.experimental.pallas import tpu as pltpu

_NUM_T = 1000

_beta = np.linspace(0.0001, 0.02, _NUM_T).astype(np.float32)
_ac = np.cumprod((1.0 - _beta).astype(np.float32), dtype=np.float32)
_TABLE_A = np.sqrt(_ac).astype(np.float32)
_TABLE_B = np.sqrt(1.0 - _ac).astype(np.float32)

_B = 128
_ROWS = 192
_COLS = 1024
_SPLIT = 2
_CH = _ROWS // _SPLIT


def _body(t_ref, ta_ref, tb_ref, x_ref, n_ref, o_ref):
    b = pl.program_id(0)
    ti = t_ref[b]
    a = ta_ref[ti]
    c = tb_ref[ti]
    o_ref[...] = a * x_ref[...] + c * n_ref[...]


def kernel(x_start, t, noise):
    xf = x_start.reshape(_B, _ROWS, _COLS)
    nf = noise.reshape(_B, _ROWS, _COLS)
    ti = t.astype(jnp.int32)
    ta = jnp.asarray(_TABLE_A)
    tb = jnp.asarray(_TABLE_B)

    grid_spec = pltpu.PrefetchScalarGridSpec(
        num_scalar_prefetch=3,
        grid=(_B, _SPLIT),
        in_specs=[
            pl.BlockSpec((1, _CH, _COLS), lambda b, j, *_: (b, j, 0)),
            pl.BlockSpec((1, _CH, _COLS), lambda b, j, *_: (b, j, 0)),
        ],
        out_specs=pl.BlockSpec((1, _CH, _COLS), lambda b, j, *_: (b, j, 0)),
    )
    out = pl.pallas_call(
        _body,
        grid_spec=grid_spec,
        out_shape=jax.ShapeDtypeStruct((_B, _ROWS, _COLS), jnp.float32),
        compiler_params=pltpu.CompilerParams(
            dimension_semantics=("parallel", "parallel"),
        ),
    )(ti, ta, tb, xf, nf)
    return out.reshape(x_start.shape)


# speedup vs baseline: 3.8480x; 3.8321x over previous
import numpy as np
import jax
import jax.numpy as jnp
from jax.experimental import pallas as pl
from jax.experimental.pallas import tpu as pltpu

_NUM_T = 1000

_beta = np.linspace(0.0001, 0.02, _NUM_T).astype(np.float32)
_ac = np.cumprod((1.0 - _beta).astype(np.float32), dtype=np.float32)
_TABLE_A = np.sqrt(_ac).astype(np.float32)
_TABLE_B = np.sqrt(1.0 - _ac).astype(np.float32)

_B = 128
_C = 3
_H = 256
_W = 256


def _body(t_ref, ta_ref, tb_ref, x_ref, n_ref, o_ref):
    b = pl.program_id(0)
    ti = t_ref[b]
    a = ta_ref[ti]
    c = tb_ref[ti]
    o_ref[...] = a * x_ref[...] + c * n_ref[...]


def kernel(x_start, t, noise):
    ti = t.astype(jnp.int32)
    ta = jnp.asarray(_TABLE_A)
    tb = jnp.asarray(_TABLE_B)

    blk = (1, _C, _H, _W)
    grid_spec = pltpu.PrefetchScalarGridSpec(
        num_scalar_prefetch=3,
        grid=(_B,),
        in_specs=[
            pl.BlockSpec(blk, lambda b, *_: (b, 0, 0, 0)),
            pl.BlockSpec(blk, lambda b, *_: (b, 0, 0, 0)),
        ],
        out_specs=pl.BlockSpec(blk, lambda b, *_: (b, 0, 0, 0)),
    )
    out = pl.pallas_call(
        _body,
        grid_spec=grid_spec,
        out_shape=jax.ShapeDtypeStruct((_B, _C, _H, _W), jnp.float32),
        compiler_params=pltpu.CompilerParams(
            dimension_semantics=("parallel",),
        ),
    )(ti, ta, tb, x_start, noise)
    return out


# speedup vs baseline: 5.1186x; 1.3302x over previous
import numpy as np
import jax
import jax.numpy as jnp
from jax.experimental import pallas as pl
from jax.experimental.pallas import tpu as pltpu

_NUM_T = 1000

_beta = np.linspace(0.0001, 0.02, _NUM_T).astype(np.float32)
_ac = np.cumprod((1.0 - _beta).astype(np.float32), dtype=np.float32)
_TABLE_A = np.sqrt(_ac).astype(np.float32)
_TABLE_B = np.sqrt(1.0 - _ac).astype(np.float32)

_B = 128
_C = 3
_H = 256
_W = 256


_BB = 2


def _body(t_ref, ta_ref, tb_ref, x_ref, n_ref, o_ref):
    g = pl.program_id(0)
    for i in range(_BB):
        ti = t_ref[g * _BB + i]
        a = ta_ref[ti]
        c = tb_ref[ti]
        o_ref[i] = a * x_ref[i] + c * n_ref[i]


def kernel(x_start, t, noise):
    ti = t.astype(jnp.int32)
    ta = jnp.asarray(_TABLE_A)
    tb = jnp.asarray(_TABLE_B)

    blk = (_BB, _C, _H, _W)
    grid_spec = pltpu.PrefetchScalarGridSpec(
        num_scalar_prefetch=3,
        grid=(_B // _BB,),
        in_specs=[
            pl.BlockSpec(blk, lambda b, *_: (b, 0, 0, 0)),
            pl.BlockSpec(blk, lambda b, *_: (b, 0, 0, 0)),
        ],
        out_specs=pl.BlockSpec(blk, lambda b, *_: (b, 0, 0, 0)),
    )
    out = pl.pallas_call(
        _body,
        grid_spec=grid_spec,
        out_shape=jax.ShapeDtypeStruct((_B, _C, _H, _W), jnp.float32),
        compiler_params=pltpu.CompilerParams(
            dimension_semantics=("parallel",),
        ),
    )(ti, ta, tb, x_start, noise)
    return out


# speedup vs baseline: 5.4293x; 1.0607x over previous
import numpy as np
import jax
import jax.numpy as jnp
from jax.experimental import pallas as pl
from jax.experimental.pallas import tpu as pltpu

_NUM_T = 1000

_beta = np.linspace(0.0001, 0.02, _NUM_T).astype(np.float32)
_ac = np.cumprod((1.0 - _beta).astype(np.float32), dtype=np.float32)
_TABLE_A = np.sqrt(_ac).astype(np.float32)
_TABLE_B = np.sqrt(1.0 - _ac).astype(np.float32)

_B = 128
_C = 3
_H = 256
_W = 256


_BB = 4


def _body(t_ref, ta_ref, tb_ref, x_ref, n_ref, o_ref):
    g = pl.program_id(0)
    for i in range(_BB):
        ti = t_ref[g * _BB + i]
        a = ta_ref[ti]
        c = tb_ref[ti]
        o_ref[i] = a * x_ref[i] + c * n_ref[i]


def kernel(x_start, t, noise):
    ti = t.astype(jnp.int32)
    ta = jnp.asarray(_TABLE_A)
    tb = jnp.asarray(_TABLE_B)

    blk = (_BB, _C, _H, _W)
    grid_spec = pltpu.PrefetchScalarGridSpec(
        num_scalar_prefetch=3,
        grid=(_B // _BB,),
        in_specs=[
            pl.BlockSpec(blk, lambda b, *_: (b, 0, 0, 0)),
            pl.BlockSpec(blk, lambda b, *_: (b, 0, 0, 0)),
        ],
        out_specs=pl.BlockSpec(blk, lambda b, *_: (b, 0, 0, 0)),
    )
    out = pl.pallas_call(
        _body,
        grid_spec=grid_spec,
        out_shape=jax.ShapeDtypeStruct((_B, _C, _H, _W), jnp.float32),
        compiler_params=pltpu.CompilerParams(
            dimension_semantics=("parallel",),
        ),
    )(ti, ta, tb, x_start, noise)
    return out
